# baseline (speedup 1.0000x reference)
import functools
import math

import jax
import jax.numpy as jnp
from jax import lax
from jax.experimental import pallas as pl
from jax.experimental.pallas import tpu as pltpu
from jax.experimental.pallas import tpu_sc as plsc

VOCAB = 50257
D_MODEL = 768
BATCH = 4
SEQ = 2048

NC = 2
NS = 16
LANES = 16
NW = NC * NS
NTOK = BATCH * SEQ
POS_PER_W = SEQ // NW
CHUNK = 32
NCH = BATCH * POS_PER_W // CHUNK
VECS_PER_ROW = D_MODEL // LANES
SCALE = math.sqrt(D_MODEL)

_mesh = plsc.VectorSubcoreMesh(core_axis_name="c", subcore_axis_name="s")


@functools.partial(
    pl.kernel,
    out_type=jax.ShapeDtypeStruct((NTOK, D_MODEL), jnp.float32),
    mesh=_mesh,
    scratch_types=[
        pltpu.VMEM((NCH, CHUNK), jnp.int32),
        pltpu.VMEM((POS_PER_W, D_MODEL), jnp.float32),
        pltpu.VMEM((CHUNK, D_MODEL), jnp.float32),
        pltpu.VMEM((CHUNK, D_MODEL), jnp.float32),
        pltpu.SemaphoreType.DMA,
        pltpu.SemaphoreType.DMA,
        pltpu.SemaphoreType.DMA,
        pltpu.SemaphoreType.DMA,
    ],
)
def _emb_kernel(ids_hbm, tok_hbm, pos_hbm, out_hbm,
                idx_v, pos_v, tok0, tok1, g0, g1, w0, w1):
    wid = lax.axis_index("s") * NC + lax.axis_index("c")
    s_base = wid * POS_PER_W
    toks = [tok0, tok1]
    gsems = [g0, g1]
    wsems = [w0, w1]

    pltpu.sync_copy(ids_hbm.at[wid], idx_v)

    gathers = [None] * NCH
    writes = [None] * NCH
    gathers[0] = pltpu.async_copy(tok_hbm.at[idx_v.at[0]], toks[0], gsems[0])
    pltpu.sync_copy(pos_hbm.at[pl.ds(s_base, POS_PER_W)], pos_v)

    for c in range(NCH):
        bu = c % 2
        b, h = c // 2, c % 2
        if c + 1 < NCH:
            if c >= 1:
                writes[c - 1].wait()
            gathers[c + 1] = pltpu.async_copy(
                tok_hbm.at[idx_v.at[c + 1]], toks[(c + 1) % 2], gsems[(c + 1) % 2])
        gathers[c].wait()

        def row_body(r, _, bu=bu, h=h):
            for l in range(VECS_PER_ROW):
                sl = pl.ds(l * LANES, LANES)
                toks[bu][r, sl] = toks[bu][r, sl] * SCALE + pos_v[h * CHUNK + r, sl]
            return _

        lax.fori_loop(0, CHUNK, row_body, 0, unroll=False)
        writes[c] = pltpu.async_copy(
            toks[bu],
            out_hbm.at[pl.ds(b * SEQ + s_base + h * CHUNK, CHUNK)],
            wsems[bu])

    writes[NCH - 2].wait()
    writes[NCH - 1].wait()


def kernel(token_ids, tok_table, pos_table):
    ids = jnp.reshape(token_ids.astype(jnp.int32), (BATCH, NW, 2, CHUNK))
    ids = jnp.transpose(ids, (1, 0, 2, 3)).reshape(NW, NCH, CHUNK)
    out = _emb_kernel(ids, tok_table, pos_table)
    return jnp.reshape(out, (BATCH, SEQ, D_MODEL))
